# Initial kernel scaffold; baseline (speedup 1.0000x reference)
import jax
import jax.numpy as jnp
from jax.experimental import pallas as pl


def kernel(tokens_one, edge_index_one, edge_type_one, tokens_two, edge_index_two, edge_type_two, emb_table, We_one, Wih_one, Whh_one, bih_one, bhh_one, bn_gamma_one, bn_beta_one, We_two, Wih_two, Whh_two, bih_two, bhh_two, bn_gamma_two, bn_beta_two, fW, fb):
    raise NotImplementedError("write your pallas kernel here")


# speedup vs baseline: 7086.1806x; 7086.1806x over previous
import jax
import jax.numpy as jnp
from jax.experimental import pallas as pl


def _head_body(b1_ref, b2_ref, fw_ref, fb_ref, out_ref):
    d = b1_ref[0, :] - b2_ref[0, :]
    euc = d * d
    logits = jnp.sum(euc[None, :] * fw_ref[:, :], axis=1) + fb_ref[0, :]
    act = jnp.where(logits >= 0, logits, 0.01 * logits)
    m = jnp.max(act)
    e = jnp.exp(act - m)
    out_ref[0, :] = e / jnp.sum(e)


def kernel(tokens_one, edge_index_one, edge_type_one, tokens_two, edge_index_two,
           edge_type_two, emb_table, We_one, Wih_one, Whh_one, bih_one, bhh_one,
           bn_gamma_one, bn_beta_one, We_two, Wih_two, Whh_two, bih_two, bhh_two,
           bn_gamma_two, bn_beta_two, fW, fb):
    b1 = bn_beta_one.reshape(1, -1)
    b2 = bn_beta_two.reshape(1, -1)
    fbr = fb.reshape(1, -1)
    out = pl.pallas_call(
        _head_body,
        out_shape=jax.ShapeDtypeStruct((1, fW.shape[0]), jnp.float32),
    )(b1, b2, fW, fbr)
    return out
